# baseline (speedup 1.0000x reference)
import functools

import jax
import jax.numpy as jnp
from jax import lax
from jax.experimental import pallas as pl
from jax.experimental.pallas import tpu as pltpu
from jax.experimental.pallas import tpu_sc as plsc

_NUM_EXPERTS = 8
_TOP_K = 2
_BM = 256
_NC, _NS = 2, 16
_NW = _NC * _NS
_CHUNK = 16


def _sc_gather(table, idx, n_rows, d):
    rows_w = n_rows // _NW
    nch = rows_w // _CHUNK

    mesh = plsc.VectorSubcoreMesh(core_axis_name="c", subcore_axis_name="s")

    @functools.partial(
        pl.kernel,
        out_type=jax.ShapeDtypeStruct((n_rows, d), jnp.float32),
        mesh=mesh,
        scratch_types=[
            pltpu.VMEM((rows_w,), jnp.int32),
            pltpu.VMEM((_CHUNK, d), jnp.float32),
            pltpu.VMEM((_CHUNK, d), jnp.float32),
            pltpu.SemaphoreType.DMA,
            pltpu.SemaphoreType.DMA,
        ],
    )
    def k(table_hbm, idx_hbm, out_hbm, idx_v, buf0, buf1, sem0, sem1):
        wid = lax.axis_index("s") * _NC + lax.axis_index("c")
        base = wid * rows_w
        pltpu.sync_copy(idx_hbm.at[pl.ds(base, rows_w)], idx_v)

        def gath(c, buf, sem):
            pltpu.async_copy(
                table_hbm.at[idx_v.at[pl.ds(c * _CHUNK, _CHUNK)]], buf, sem)

        def drain(buf, sem):
            pltpu.make_async_copy(
                table_hbm.at[idx_v.at[pl.ds(0, _CHUNK)]], buf, sem).wait()

        def put(c, buf):
            pltpu.sync_copy(buf, out_hbm.at[pl.ds(base + c * _CHUNK, _CHUNK)])

        gath(0, buf0, sem0)

        def body(i, _):
            e = 2 * i
            gath(e + 1, buf1, sem1)
            drain(buf0, sem0)
            put(e, buf0)

            @pl.when(e + 2 < nch)
            def _():
                gath(e + 2, buf0, sem0)

            drain(buf1, sem1)
            put(e + 1, buf1)
            return 0

        lax.fori_loop(0, nch // 2, body, 0)

    return k(table, idx)


def _matmul_block(be_ref, x_ref, w_ref, b_ref, s_ref, o_ref):
    x = x_ref[...]
    w = w_ref[0]
    y = lax.dot_general(x, w, (((1,), (1,)), ((), ())),
                        preferred_element_type=jnp.float32)
    y = y + b_ref[0]
    o_ref[...] = y * s_ref[...]


def _grouped_matmul(dispatch, W, b, w_col, block_expert, num_blocks, d):
    grid_spec = pltpu.PrefetchScalarGridSpec(
        num_scalar_prefetch=1,
        grid=(num_blocks,),
        in_specs=[
            pl.BlockSpec((_BM, d), lambda i, be: (i, 0)),
            pl.BlockSpec((1, d, d), lambda i, be: (be[i], 0, 0)),
            pl.BlockSpec((1, 1, d), lambda i, be: (be[i], 0, 0)),
            pl.BlockSpec((_BM, 1), lambda i, be: (i, 0)),
        ],
        out_specs=pl.BlockSpec((_BM, d), lambda i, be: (i, 0)),
    )
    return pl.pallas_call(
        _matmul_block,
        grid_spec=grid_spec,
        out_shape=jax.ShapeDtypeStruct((num_blocks * _BM, d), jnp.float32),
    )(block_expert, dispatch, W, b.reshape(b.shape[0], 1, d), w_col)


def _add_block(a_ref, b_ref, o_ref):
    o_ref[...] = a_ref[...] + b_ref[...]


def _pair_add(gcat, n_tok, d):
    nb = n_tok // _BM
    return pl.pallas_call(
        _add_block,
        grid=(nb,),
        in_specs=[
            pl.BlockSpec((_BM, d), lambda i: (i, 0)),
            pl.BlockSpec((_BM, d), lambda i: (i + nb, 0)),
        ],
        out_specs=pl.BlockSpec((_BM, d), lambda i: (i, 0)),
        out_shape=jax.ShapeDtypeStruct((n_tok, d), jnp.float32),
    )(gcat, gcat)


def kernel(hidden, gate_logits, W, b):
    bsz, seq, d = hidden.shape
    n_tok = bsz * seq
    k = _TOP_K
    e = _NUM_EXPERTS
    n_slots = n_tok * k
    p = n_slots + e * _BM
    num_blocks = p // _BM

    hidden_flat = hidden.reshape(n_tok, d)

    probs = jax.nn.softmax(gate_logits, axis=-1)
    topk_w, topk_i = lax.top_k(probs, k)
    flat_e = topk_i.reshape(-1)

    onehot = (flat_e[:, None] == jnp.arange(e)[None, :]).astype(jnp.int32)
    cum = jnp.cumsum(onehot, axis=0)
    rank = jnp.take_along_axis(cum, flat_e[:, None], axis=1)[:, 0] - 1
    counts = cum[-1]
    padded_counts = ((counts + _BM - 1) // _BM) * _BM
    padded_end = jnp.cumsum(padded_counts)
    padded_start = padded_end - padded_counts
    padded_pos = padded_start[flat_e] + rank

    block_expert = jnp.minimum(
        jnp.sum(jnp.arange(num_blocks)[:, None] * _BM >= padded_end[None, :],
                axis=1), e - 1).astype(jnp.int32)

    tok_of_slot = jnp.arange(n_slots, dtype=jnp.int32) // k
    gather_tok = jnp.zeros((p,), jnp.int32).at[padded_pos].set(tok_of_slot)
    w_col = jnp.zeros((p,), jnp.float32).at[padded_pos].set(
        topk_w.reshape(-1)).reshape(p, 1)

    dispatch = _sc_gather(hidden_flat, gather_tok, p, d)

    y = _grouped_matmul(dispatch, W, b, w_col, block_expert, num_blocks, d)

    pos = padded_pos.reshape(n_tok, k)
    cat_pos = jnp.concatenate([pos[:, 0], pos[:, 1]])
    gcat = _sc_gather(y, cat_pos, n_slots, d)
    combined = _pair_add(gcat, n_tok, d)
    return combined.reshape(bsz, seq, d)


# speedup vs baseline: 1.6946x; 1.6946x over previous
import functools

import jax
import jax.numpy as jnp
from jax import lax
from jax.experimental import pallas as pl
from jax.experimental.pallas import tpu as pltpu
from jax.experimental.pallas import tpu_sc as plsc

_NUM_EXPERTS = 8
_TOP_K = 2
_BM = 256
_NC, _NS = 2, 16
_NW = _NC * _NS
_CHUNK = 16


def _sc_dispatch(hidden_flat, pos0, pos1, n_tok, p, d):
    tok_w = n_tok // _NW
    nch = tok_w // _CHUNK

    mesh = plsc.VectorSubcoreMesh(core_axis_name="c", subcore_axis_name="s")

    @functools.partial(
        pl.kernel,
        out_type=jax.ShapeDtypeStruct((p, d), jnp.float32),
        mesh=mesh,
        scratch_types=[
            pltpu.VMEM((nch, _CHUNK), jnp.int32),
            pltpu.VMEM((nch, _CHUNK), jnp.int32),
            pltpu.VMEM((_CHUNK, d), jnp.float32),
            pltpu.VMEM((_CHUNK, d), jnp.float32),
            pltpu.SemaphoreType.DMA,
            pltpu.SemaphoreType.DMA,
            pltpu.SemaphoreType.DMA,
        ],
    )
    def k(hid_hbm, p0_hbm, p1_hbm, out_hbm, i0_v, i1_v, buf0, buf1,
          sem0, sem1, semw):
        wid = lax.axis_index("s") * _NC + lax.axis_index("c")
        base = wid * tok_w
        pltpu.sync_copy(p0_hbm.at[wid], i0_v)
        pltpu.sync_copy(p1_hbm.at[wid], i1_v)

        def rd(c, buf, sem):
            pltpu.async_copy(hid_hbm.at[pl.ds(base + c * _CHUNK, _CHUNK)],
                             buf, sem)

        def rd_wait(buf, sem):
            pltpu.make_async_copy(hid_hbm.at[pl.ds(0, _CHUNK)], buf,
                                  sem).wait()

        def wr(c, buf):
            pltpu.async_copy(buf, out_hbm.at[i0_v.at[c]], semw)
            pltpu.async_copy(buf, out_hbm.at[i1_v.at[c]], semw)

        def wr_wait(buf):
            pltpu.make_async_copy(buf, out_hbm.at[i0_v.at[0]], semw).wait()
            pltpu.make_async_copy(buf, out_hbm.at[i0_v.at[0]], semw).wait()

        rd(0, buf0, sem0)

        def body(i, _):
            e = 2 * i
            rd(e + 1, buf1, sem1)
            rd_wait(buf0, sem0)
            wr(e, buf0)
            wr_wait(buf0)

            @pl.when(e + 2 < nch)
            def _():
                rd(e + 2, buf0, sem0)

            rd_wait(buf1, sem1)
            wr(e + 1, buf1)
            wr_wait(buf1)
            return 0

        lax.fori_loop(0, nch // 2, body, 0)

    return k(hidden_flat, pos0.reshape(_NW, nch, _CHUNK),
             pos1.reshape(_NW, nch, _CHUNK))


def _sc_gather(table, idx, n_rows, d):
    rows_w = n_rows // _NW
    nch = rows_w // _CHUNK

    mesh = plsc.VectorSubcoreMesh(core_axis_name="c", subcore_axis_name="s")

    @functools.partial(
        pl.kernel,
        out_type=jax.ShapeDtypeStruct((n_rows, d), jnp.float32),
        mesh=mesh,
        scratch_types=[
            pltpu.VMEM((rows_w,), jnp.int32),
            pltpu.VMEM((_CHUNK, d), jnp.float32),
            pltpu.VMEM((_CHUNK, d), jnp.float32),
            pltpu.SemaphoreType.DMA,
            pltpu.SemaphoreType.DMA,
        ],
    )
    def k(table_hbm, idx_hbm, out_hbm, idx_v, buf0, buf1, sem0, sem1):
        wid = lax.axis_index("s") * _NC + lax.axis_index("c")
        base = wid * rows_w
        pltpu.sync_copy(idx_hbm.at[pl.ds(base, rows_w)], idx_v)

        def gath(c, buf, sem):
            pltpu.async_copy(
                table_hbm.at[idx_v.at[pl.ds(c * _CHUNK, _CHUNK)]], buf, sem)

        def drain(buf, sem):
            pltpu.make_async_copy(
                table_hbm.at[idx_v.at[pl.ds(0, _CHUNK)]], buf, sem).wait()

        def put(c, buf):
            pltpu.sync_copy(buf, out_hbm.at[pl.ds(base + c * _CHUNK, _CHUNK)])

        gath(0, buf0, sem0)

        def body(i, _):
            e = 2 * i
            gath(e + 1, buf1, sem1)
            drain(buf0, sem0)
            put(e, buf0)

            @pl.when(e + 2 < nch)
            def _():
                gath(e + 2, buf0, sem0)

            drain(buf1, sem1)
            put(e + 1, buf1)
            return 0

        lax.fori_loop(0, nch // 2, body, 0)

    return k(table, idx)


def _matmul_block(be_ref, x_ref, w_ref, b_ref, o_ref):
    x = x_ref[...]
    w = w_ref[0]
    y = lax.dot_general(x, w, (((1,), (1,)), ((), ())),
                        preferred_element_type=jnp.float32)
    o_ref[...] = y + b_ref[0]


def _grouped_matmul(dispatch, W, b, block_expert, num_blocks, d):
    grid_spec = pltpu.PrefetchScalarGridSpec(
        num_scalar_prefetch=1,
        grid=(num_blocks,),
        in_specs=[
            pl.BlockSpec((_BM, d), lambda i, be: (i, 0)),
            pl.BlockSpec((1, d, d), lambda i, be: (be[i], 0, 0)),
            pl.BlockSpec((1, 1, d), lambda i, be: (be[i], 0, 0)),
        ],
        out_specs=pl.BlockSpec((_BM, d), lambda i, be: (i, 0)),
    )
    return pl.pallas_call(
        _matmul_block,
        grid_spec=grid_spec,
        out_shape=jax.ShapeDtypeStruct((num_blocks * _BM, d), jnp.float32),
    )(block_expert, dispatch, W, b.reshape(b.shape[0], 1, d))


def _wadd_block(a_ref, b_ref, wa_ref, wb_ref, o_ref):
    o_ref[...] = a_ref[...] * wa_ref[...] + b_ref[...] * wb_ref[...]


def _weighted_pair_add(gcat, w0, w1, n_tok, d):
    nb = n_tok // _BM
    return pl.pallas_call(
        _wadd_block,
        grid=(nb,),
        in_specs=[
            pl.BlockSpec((_BM, d), lambda i: (i, 0)),
            pl.BlockSpec((_BM, d), lambda i: (i + nb, 0)),
            pl.BlockSpec((_BM, 1), lambda i: (i, 0)),
            pl.BlockSpec((_BM, 1), lambda i: (i, 0)),
        ],
        out_specs=pl.BlockSpec((_BM, d), lambda i: (i, 0)),
        out_shape=jax.ShapeDtypeStruct((n_tok, d), jnp.float32),
    )(gcat, gcat, w0, w1)


def kernel(hidden, gate_logits, W, b):
    bsz, seq, d = hidden.shape
    n_tok = bsz * seq
    k = _TOP_K
    e = _NUM_EXPERTS
    n_slots = n_tok * k
    p = n_slots + e * _BM
    num_blocks = p // _BM

    hidden_flat = hidden.reshape(n_tok, d)

    probs = jax.nn.softmax(gate_logits, axis=-1)
    topk_w, topk_i = lax.top_k(probs, k)
    flat_e = topk_i.reshape(-1)

    onehot = (flat_e[:, None] == jnp.arange(e)[None, :]).astype(jnp.int32)
    cum = jnp.cumsum(onehot, axis=0)
    rank = jnp.sum(onehot * cum, axis=1) - 1
    counts = cum[-1]
    padded_counts = ((counts + _BM - 1) // _BM) * _BM
    padded_end = jnp.cumsum(padded_counts)
    padded_start = padded_end - padded_counts
    seg_base = jnp.sum(onehot * padded_start[None, :], axis=1)
    padded_pos = (seg_base + rank).astype(jnp.int32)

    block_expert = jnp.minimum(
        jnp.sum(jnp.arange(num_blocks)[:, None] * _BM >= padded_end[None, :],
                axis=1), e - 1).astype(jnp.int32)

    pos = padded_pos.reshape(n_tok, k)
    pos0, pos1 = pos[:, 0], pos[:, 1]

    dispatch = _sc_dispatch(hidden_flat, pos0, pos1, n_tok, p, d)

    y = _grouped_matmul(dispatch, W, b, block_expert, num_blocks, d)

    cat_pos = jnp.concatenate([pos0, pos1])
    gcat = _sc_gather(y, cat_pos, n_slots, d)
    combined = _weighted_pair_add(gcat, topk_w[:, :1], topk_w[:, 1:],
                                  n_tok, d)
    return combined.reshape(bsz, seq, d)
